# Initial kernel scaffold; baseline (speedup 1.0000x reference)
import jax
import jax.numpy as jnp
from jax.experimental import pallas as pl


def kernel(x, edge_index, W1_l, b1, W1_r, W2_l, b2, W2_r):
    raise NotImplementedError("write your pallas kernel here")


# speedup vs baseline: 9.9005x; 9.9005x over previous
import functools

import jax
import jax.numpy as jnp
from jax import lax
from jax.experimental import pallas as pl
from jax.experimental.pallas import tpu as pltpu
from jax.experimental.pallas import tpu_sc as plsc

N = 10000
D = 128
E = 320000
NC = 2
NS = 16
NW = NC * NS
EPW = E // NW
CH = 80
NCHUNK = EPW // CH
NP_ = 10240
RPT = NP_ // NS
R = 1000


def _make_agg():
    mesh = plsc.VectorSubcoreMesh(core_axis_name="c", subcore_axis_name="s")

    @functools.partial(
        pl.kernel, mesh=mesh,
        out_type=jax.ShapeDtypeStruct((NC * NP_, D), jnp.float32),
        scratch_types=[
            pltpu.VMEM((8, CH), jnp.int32),
            pltpu.VMEM((2, CH, D), jnp.float32),
            pltpu.VMEM_SHARED((NP_, D), jnp.float32),
            pltpu.VMEM((16,), jnp.int32),
            pltpu.SemaphoreType.DMA,
            pltpu.SemaphoreType.DMA,
            pltpu.SemaphoreType.DMA,
            pltpu.SemaphoreType.DMA,
            pltpu.SemaphoreType.DMA,
            pltpu.SemaphoreType.DMA,
        ])
    def agg(tbl, edges, zeros, fill, flag, part_o,
            ring, rows2, acc, fsm, g0, g1, i0, i1, i2, i3):
        gsem = (g0, g1)
        isem = (i0, i1, i2, i3)

        c = lax.axis_index("c")
        s = lax.axis_index("s")
        wid = c * NS + s
        r0 = s * RPT

        pltpu.sync_copy(zeros.at[pl.ds(r0, RPT)], acc.at[pl.ds(r0, RPT)])
        pltpu.sync_copy(flag, fsm)
        pltpu.sync_copy(fill, rows2)
        f = fsm[...][0]
        plsc.subcore_barrier()

        def idx_load(chunk, slot, sem):
            return pltpu.async_copy(edges.at[wid, chunk],
                                    ring.at[pl.ds(2 * slot, 2)], sem)

        def gather(chunk_slot, buf):
            return pltpu.async_copy(tbl.at[ring.at[2 * chunk_slot]],
                                    rows2.at[buf], gsem[buf])

        idx_load(0, 0, isem[0]).wait()
        idx_load(1, 1, isem[1]).wait()

        @pl.when(f == 1)
        def _():
            gather(0, 0)
            gather(1, 1)

        idx_load(2, 2, isem[2])
        idx_load(3, 3, isem[3])

        def step(i, r):
            b = r % 2

            @pl.when(f == 1)
            def _():
                pltpu.make_async_copy(tbl.at[ring.at[2 * r]], rows2.at[b],
                                      gsem[b]).wait()

            pltpu.sync_copy(rows2.at[b], acc.at[ring.at[2 * r + 1]], add=True)
            nslot = (r + 2) % 4

            @pl.when(i + 2 < NCHUNK)
            def _():
                pltpu.make_async_copy(edges.at[wid, i + 2],
                                      ring.at[pl.ds(2 * nslot, 2)],
                                      isem[nslot]).wait()

                @pl.when(f == 1)
                def _():
                    gather(nslot, b)

            @pl.when(i + 4 < NCHUNK)
            def _():
                idx_load(i + 4, r, isem[r])

        def quad(j, carry):
            for r in range(4):
                step(4 * j + r, r)
            return carry
        lax.fori_loop(0, (NCHUNK - 1) // 4, quad, 0)
        step(jnp.int32(NCHUNK - 1), (NCHUNK - 1) % 4)

        plsc.subcore_barrier()
        o0 = c * NP_ + r0
        pltpu.sync_copy(acc.at[pl.ds(r0, RPT)], part_o.at[pl.ds(o0, RPT)])

    return agg


_agg = _make_agg()

_DN = (((1,), (1,)), ((), ()))


def _linear(mean, x, wl_ref, b_ref, wr_ref):
    return (lax.dot_general(mean, wl_ref[...], _DN,
                            preferred_element_type=jnp.float32,
                            precision=lax.Precision.HIGHEST)
            + b_ref[...]
            + lax.dot_general(x, wr_ref[...], _DN,
                              preferred_element_type=jnp.float32,
                              precision=lax.Precision.HIGHEST))


def _layer1_body(p_ref, q_ref, x_ref, wl_ref, b_ref, wr_ref, h_ref, c_ref):
    psum = p_ref[0] + p_ref[1]
    cnt = (q_ref[0] + q_ref[1])[:, 0:1]
    mean = psum / jnp.maximum(cnt, 1.0)
    y = _linear(mean, x_ref[...], wl_ref, b_ref, wr_ref)
    h_ref[...] = jnp.where(y >= 0, y, 0.01 * y)
    c_ref[...] = cnt


_layer1 = pl.pallas_call(
    _layer1_body,
    grid=(N // R,),
    in_specs=[
        pl.BlockSpec((2, R, D), lambda i: (0, i, 0)),
        pl.BlockSpec((2, R, D), lambda i: (0, i, 0)),
        pl.BlockSpec((R, D), lambda i: (i, 0)),
        pl.BlockSpec((D, D), lambda i: (0, 0)),
        pl.BlockSpec((1, D), lambda i: (0, 0)),
        pl.BlockSpec((D, D), lambda i: (0, 0)),
    ],
    out_specs=(pl.BlockSpec((R, D), lambda i: (i, 0)),
               pl.BlockSpec((R, 1), lambda i: (i, 0))),
    out_shape=(jax.ShapeDtypeStruct((N, D), jnp.float32),
               jax.ShapeDtypeStruct((N, 1), jnp.float32)),
)


def _layer2_body(p_ref, c_ref, x_ref, wl_ref, b_ref, wr_ref, o_ref):
    psum = p_ref[0] + p_ref[1]
    mean = psum / jnp.maximum(c_ref[...], 1.0)
    o_ref[...] = _linear(mean, x_ref[...], wl_ref, b_ref, wr_ref)


_layer2 = pl.pallas_call(
    _layer2_body,
    grid=(N // R,),
    in_specs=[
        pl.BlockSpec((2, R, D), lambda i: (0, i, 0)),
        pl.BlockSpec((R, 1), lambda i: (i, 0)),
        pl.BlockSpec((R, D), lambda i: (i, 0)),
        pl.BlockSpec((D, D), lambda i: (0, 0)),
        pl.BlockSpec((1, D), lambda i: (0, 0)),
        pl.BlockSpec((D, D), lambda i: (0, 0)),
    ],
    out_specs=pl.BlockSpec((R, D), lambda i: (i, 0)),
    out_shape=jax.ShapeDtypeStruct((N, D), jnp.float32),
)


def kernel(x, edge_index, W1_l, b1, W1_r, W2_l, b2, W2_r):
    ei = edge_index.astype(jnp.int32)
    edges = (ei.reshape(2, NW, NCHUNK, CH).transpose(1, 2, 0, 3)
             .reshape(NW, NCHUNK, 2, CH))
    z128 = jnp.zeros((NP_, D), jnp.float32)
    fill = jnp.ones((2, CH, D), jnp.float32)
    f_gather = jnp.ones((16,), jnp.int32)
    f_count = jnp.zeros((16,), jnp.int32)

    q = _agg(x, edges, z128, fill, f_count).reshape(NC, NP_, D)
    p1 = _agg(x, edges, z128, fill, f_gather).reshape(NC, NP_, D)
    h, cnt = _layer1(p1, q, x, W1_l, b1.reshape(1, D), W1_r)

    p2 = _agg(h, edges, z128, fill, f_gather).reshape(NC, NP_, D)
    out = _layer2(p2, cnt, h, W2_l, b2.reshape(1, D), W2_r)
    return out
